# Initial kernel scaffold; baseline (speedup 1.0000x reference)
import functools

import jax
import jax.numpy as jnp
from jax import lax
from jax.experimental import pallas as pl
from jax.experimental.pallas import tpu as pltpu
from jax.experimental.pallas import tpu_sc as plsc

N = 10000
E = 320000
D = 128
NC = 2
NS = 16
NW = NC * NS
PER_W = E // NW
CH = 80
NCH = PER_W // CH
ROWS_PER_TILE = N // NS


def _sc_segsum(with_counts):
    mesh = plsc.VectorSubcoreMesh(core_axis_name="c", subcore_axis_name="s")
    out_type = [jax.ShapeDtypeStruct((NC, N, D), jnp.float32)]
    scratch = [
        pltpu.VMEM_SHARED((N, D), jnp.float32),
        pltpu.VMEM((NCH, CH), jnp.int32),
        pltpu.VMEM((NCH, CH), jnp.int32),
        pltpu.VMEM((CH, D), jnp.float32),
    ]
    if with_counts:
        out_type.append(jax.ShapeDtypeStruct((NC, N, 16), jnp.float32))
        scratch.append(pltpu.VMEM_SHARED((N, 16), jnp.float32))
        scratch.append(pltpu.VMEM((CH, 16), jnp.float32))

    def body(*refs):
        if with_counts:
            (x_hbm, src_hbm, dst_hbm, zacc_hbm, zcnt_hbm, ones_hbm,
             acc_out, cnt_out, acc_sh, src_v, dst_v, rows_v,
             cnt_sh, ones_v) = refs
        else:
            (x_hbm, src_hbm, dst_hbm, zacc_hbm,
             acc_out, acc_sh, src_v, dst_v, rows_v) = refs

        cid = lax.axis_index("c")
        sid = lax.axis_index("s")
        wid = cid * NS + sid
        row0 = sid * ROWS_PER_TILE

        pltpu.sync_copy(src_hbm.at[wid], src_v)
        pltpu.sync_copy(dst_hbm.at[wid], dst_v)
        pltpu.sync_copy(zacc_hbm.at[pl.ds(row0, ROWS_PER_TILE)],
                        acc_sh.at[pl.ds(row0, ROWS_PER_TILE)])
        if with_counts:
            pltpu.sync_copy(zcnt_hbm.at[pl.ds(row0, ROWS_PER_TILE)],
                            cnt_sh.at[pl.ds(row0, ROWS_PER_TILE)])
            pltpu.sync_copy(ones_hbm, ones_v)
        plsc.subcore_barrier()

        @pl.loop(0, NCH)
        def _(i):
            pltpu.sync_copy(x_hbm.at[src_v.at[i]], rows_v)
            pltpu.sync_copy(rows_v, acc_sh.at[dst_v.at[i]], add=True)
            if with_counts:
                pltpu.sync_copy(ones_v, cnt_sh.at[dst_v.at[i]], add=True)

        plsc.subcore_barrier()

        pltpu.sync_copy(acc_sh.at[pl.ds(row0, ROWS_PER_TILE)],
                        acc_out.at[cid].at[pl.ds(row0, ROWS_PER_TILE)])
        if with_counts:
            pltpu.sync_copy(cnt_sh.at[pl.ds(row0, ROWS_PER_TILE)],
                            cnt_out.at[cid].at[pl.ds(row0, ROWS_PER_TILE)])

    return pl.kernel(body, out_type=tuple(out_type), mesh=mesh,
                     scratch_types=scratch)


def _tc_layer(relu):
    B = 1000

    def body(acc_ref, cnt_ref, x_ref, wl_ref, b_ref, wr_ref, o_ref):
        s = acc_ref[0] + acc_ref[1]
        cnt = cnt_ref[0, :, 0] + cnt_ref[1, :, 0]
        mean = s * (1.0 / jnp.maximum(cnt, 1.0))[:, None]
        dn = (((1,), (1,)), ((), ()))
        r = (lax.dot_general(mean, wl_ref[...], dn,
                             preferred_element_type=jnp.float32)
             + lax.dot_general(x_ref[...], wr_ref[...], dn,
                               preferred_element_type=jnp.float32)
             + b_ref[...])
        o_ref[...] = jnp.maximum(r, 0.0) if relu else r

    return pl.pallas_call(
        body,
        grid=(N // B,),
        in_specs=[
            pl.BlockSpec((NC, B, D), lambda i: (0, i, 0)),
            pl.BlockSpec((NC, B, 16), lambda i: (0, i, 0)),
            pl.BlockSpec((B, D), lambda i: (i, 0)),
            pl.BlockSpec((D, D), lambda i: (0, 0)),
            pl.BlockSpec((1, D), lambda i: (0, 0)),
            pl.BlockSpec((D, D), lambda i: (0, 0)),
        ],
        out_specs=pl.BlockSpec((B, D), lambda i: (i, 0)),
        out_shape=jax.ShapeDtypeStruct((N, D), jnp.float32),
    )


@jax.jit
def kernel(x, edge_index, W1_l, b1_l, W1_r, W2_l, b2_l, W2_r):
    ei = edge_index.astype(jnp.int32)
    src = ei[0].reshape(NW, NCH, CH)
    dst = ei[1].reshape(NW, NCH, CH)
    zacc = jnp.zeros((N, D), jnp.float32)
    zcnt = jnp.zeros((N, 16), jnp.float32)
    ones = jnp.ones((CH, 16), jnp.float32)
    b1 = b1_l.reshape(1, D)
    b2 = b2_l.reshape(1, D)

    acc1, cntp = _sc_segsum(True)(x, src, dst, zacc, zcnt, ones)
    h = _tc_layer(True)(acc1, cntp, x, W1_l, b1, W1_r)
    acc2 = _sc_segsum(False)(h, src, dst, zacc)
    out = _tc_layer(False)(acc2, cntp, h, W2_l, b2, W2_r)
    return out


# speedup vs baseline: 6.9690x; 6.9690x over previous
import functools

import jax
import jax.numpy as jnp
from jax import lax
from jax.experimental import pallas as pl
from jax.experimental.pallas import tpu as pltpu
from jax.experimental.pallas import tpu_sc as plsc

N = 10000
E = 320000
D = 128
NC = 2
NS = 16
NW = NC * NS
PER_W = E // NW
CH = 80
NCH = PER_W // CH
ROWS_A = 624
TAIL0 = NS * ROWS_A
TAIL = N - TAIL0


def _sc_segsum():
    mesh = plsc.VectorSubcoreMesh(core_axis_name="c", subcore_axis_name="s")
    out_type = jax.ShapeDtypeStruct((NC, N, D), jnp.float32)
    scratch = [
        pltpu.VMEM_SHARED((N, D), jnp.float32),
        pltpu.VMEM((NCH, CH), jnp.int32),
        pltpu.VMEM((NCH, CH), jnp.int32),
        pltpu.VMEM((CH, D), jnp.float32),
    ]

    def body(x_hbm, src_hbm, dst_hbm, zacc_hbm, acc_out,
             acc_sh, src_v, dst_v, rows_v):
        cid = lax.axis_index("c")
        sid = lax.axis_index("s")
        wid = cid * NS + sid
        row0 = sid * ROWS_A

        pltpu.sync_copy(src_hbm.at[wid], src_v)
        pltpu.sync_copy(dst_hbm.at[wid], dst_v)
        pltpu.sync_copy(zacc_hbm.at[pl.ds(row0, ROWS_A)],
                        acc_sh.at[pl.ds(row0, ROWS_A)])

        @pl.when(sid == NS - 1)
        def _():
            pltpu.sync_copy(zacc_hbm.at[pl.ds(TAIL0, TAIL)],
                            acc_sh.at[pl.ds(TAIL0, TAIL)])

        plsc.subcore_barrier()

        @pl.loop(0, NCH)
        def _(i):
            pltpu.sync_copy(x_hbm.at[src_v.at[i]], rows_v)
            pltpu.sync_copy(rows_v, acc_sh.at[dst_v.at[i]], add=True)

        plsc.subcore_barrier()

        pltpu.sync_copy(acc_sh.at[pl.ds(row0, ROWS_A)],
                        acc_out.at[cid].at[pl.ds(row0, ROWS_A)])

        @pl.when(sid == NS - 1)
        def _():
            pltpu.sync_copy(acc_sh.at[pl.ds(TAIL0, TAIL)],
                            acc_out.at[cid].at[pl.ds(TAIL0, TAIL)])

    return pl.kernel(body, out_type=out_type, mesh=mesh,
                     scratch_types=scratch)


def _sc_counts(W=128):
    mesh = plsc.VectorSubcoreMesh(core_axis_name="c", subcore_axis_name="s")
    out_type = jax.ShapeDtypeStruct((NC, N, W), jnp.float32)
    scratch = [
        pltpu.VMEM_SHARED((N, W), jnp.float32),
        pltpu.VMEM((NCH, CH), jnp.int32),
        pltpu.VMEM((CH, W), jnp.float32),
    ]

    def body(dst_hbm, zcnt_hbm, ones_hbm, cnt_out, cnt_sh, dst_v, ones_v):
        cid = lax.axis_index("c")
        sid = lax.axis_index("s")
        wid = cid * NS + sid
        row0 = sid * ROWS_A

        pltpu.sync_copy(dst_hbm.at[wid], dst_v)
        pltpu.sync_copy(ones_hbm, ones_v)
        pltpu.sync_copy(zcnt_hbm.at[pl.ds(row0, ROWS_A)],
                        cnt_sh.at[pl.ds(row0, ROWS_A)])

        @pl.when(sid == NS - 1)
        def _():
            pltpu.sync_copy(zcnt_hbm.at[pl.ds(TAIL0, TAIL)],
                            cnt_sh.at[pl.ds(TAIL0, TAIL)])

        plsc.subcore_barrier()

        @pl.loop(0, NCH)
        def _(i):
            pltpu.sync_copy(ones_v, cnt_sh.at[dst_v.at[i]], add=True)

        plsc.subcore_barrier()

        pltpu.sync_copy(cnt_sh.at[pl.ds(row0, ROWS_A)],
                        cnt_out.at[cid].at[pl.ds(row0, ROWS_A)])

        @pl.when(sid == NS - 1)
        def _():
            pltpu.sync_copy(cnt_sh.at[pl.ds(TAIL0, TAIL)],
                            cnt_out.at[cid].at[pl.ds(TAIL0, TAIL)])

    return pl.kernel(body, out_type=out_type, mesh=mesh,
                     scratch_types=scratch)


def _tc_layer(relu):
    B = 1000

    def body(acc_ref, cnt_ref, x_ref, wl_ref, b_ref, wr_ref, o_ref):
        s = acc_ref[0] + acc_ref[1]
        cnt = cnt_ref[0, :, 0] + cnt_ref[1, :, 0]
        mean = s * (1.0 / jnp.maximum(cnt, 1.0))[:, None]
        dn = (((1,), (1,)), ((), ()))
        r = (lax.dot_general(mean, wl_ref[...], dn,
                             preferred_element_type=jnp.float32)
             + lax.dot_general(x_ref[...], wr_ref[...], dn,
                               preferred_element_type=jnp.float32)
             + b_ref[...])
        o_ref[...] = jnp.maximum(r, 0.0) if relu else r

    return pl.pallas_call(
        body,
        grid=(N // B,),
        in_specs=[
            pl.BlockSpec((NC, B, D), lambda i: (0, i, 0)),
            pl.BlockSpec((NC, B, 128), lambda i: (0, i, 0)),
            pl.BlockSpec((B, D), lambda i: (i, 0)),
            pl.BlockSpec((D, D), lambda i: (0, 0)),
            pl.BlockSpec((1, D), lambda i: (0, 0)),
            pl.BlockSpec((D, D), lambda i: (0, 0)),
        ],
        out_specs=pl.BlockSpec((B, D), lambda i: (i, 0)),
        out_shape=jax.ShapeDtypeStruct((N, D), jnp.float32),
    )


@jax.jit
def kernel(x, edge_index, W1_l, b1_l, W1_r, W2_l, b2_l, W2_r):
    ei = edge_index.astype(jnp.int32)
    src = ei[0].reshape(NW, NCH, CH)
    dst = ei[1].reshape(NW, NCH, CH)
    zacc = jnp.zeros((N, D), jnp.float32)
    zcnt = jnp.zeros((N, 128), jnp.float32)
    ones = jnp.ones((CH, 128), jnp.float32)
    b1 = b1_l.reshape(1, D)
    b2 = b2_l.reshape(1, D)

    cntp = _sc_counts()(dst, zcnt, ones)
    acc1 = _sc_segsum()(x, src, dst, zacc)
    h = _tc_layer(True)(acc1, cntp, x, W1_l, b1, W1_r)
    acc2 = _sc_segsum()(h, src, dst, zacc)
    out = _tc_layer(False)(acc2, cntp, h, W2_l, b2, W2_r)
    return out


# speedup vs baseline: 10.0430x; 1.4411x over previous
import dataclasses

import jax
import jax.numpy as jnp
from jax import lax
from jax.experimental import pallas as pl
from jax.experimental.pallas import tpu as pltpu
from jax.experimental.pallas import tpu_sc as plsc

N = 10000
E = 320000
D = 128
NC = 2
NS = 16
NW = NC * NS
PER_W = E // NW
CH = 80
NCH = PER_W // CH
ROWS_A = 624
TAIL0 = NS * ROWS_A
TAIL = N - TAIL0


def _sc_segsum():
    mesh = plsc.VectorSubcoreMesh(core_axis_name="c", subcore_axis_name="s")
    out_type = jax.ShapeDtypeStruct((NC, N, D), jnp.float32)
    scratch = [
        pltpu.VMEM_SHARED((N, D), jnp.float32),
        pltpu.VMEM((PER_W,), jnp.int32),
        pltpu.VMEM((NCH, CH), jnp.int32),
        pltpu.VMEM((CH, D), jnp.float32),
        pltpu.VMEM((CH, D), jnp.float32),
        pltpu.SemaphoreType.DMA,
        pltpu.SemaphoreType.DMA,
    ]

    def body(x_hbm, src_hbm, dst_hbm, zacc_hbm, acc_out,
             acc_sh, src_v, dst_v, rows_a, rows_b, sem_a, sem_b):
        cid = lax.axis_index("c")
        sid = lax.axis_index("s")
        wid = cid * NS + sid
        row0 = sid * ROWS_A

        pltpu.sync_copy(src_hbm.at[wid], src_v)
        pltpu.sync_copy(dst_hbm.at[wid], dst_v)
        pltpu.sync_copy(zacc_hbm.at[pl.ds(row0, ROWS_A)],
                        acc_sh.at[pl.ds(row0, ROWS_A)])

        @pl.when(sid == NS - 1)
        def _():
            pltpu.sync_copy(zacc_hbm.at[pl.ds(TAIL0, TAIL)],
                            acc_sh.at[pl.ds(TAIL0, TAIL)])

        plsc.subcore_barrier()

        def start_gather(c, buf, sem):
            pltpu.async_copy(x_hbm.at[src_v.at[pl.ds(c * CH, CH)]], buf, sem)

        def wait_gather(c, buf, sem):
            pltpu.make_async_copy(
                x_hbm.at[src_v.at[pl.ds(c * CH, CH)]], buf, sem).wait()

        def scatter(c, buf):
            pltpu.sync_copy(buf, acc_sh.at[dst_v.at[c]], add=True)

        start_gather(0, rows_a, sem_a)

        @pl.loop(0, (NCH - 1) // 2)
        def _(k):
            c0 = 2 * k
            c1 = c0 + 1
            wait_gather(c0, rows_a, sem_a)
            start_gather(c1, rows_b, sem_b)
            scatter(c0, rows_a)
            wait_gather(c1, rows_b, sem_b)
            start_gather(c0 + 2, rows_a, sem_a)
            scatter(c1, rows_b)

        wait_gather(NCH - 1, rows_a, sem_a)
        scatter(NCH - 1, rows_a)

        plsc.subcore_barrier()

        pltpu.sync_copy(acc_sh.at[pl.ds(row0, ROWS_A)],
                        acc_out.at[cid].at[pl.ds(row0, ROWS_A)])

        @pl.when(sid == NS - 1)
        def _():
            pltpu.sync_copy(acc_sh.at[pl.ds(TAIL0, TAIL)],
                            acc_out.at[cid].at[pl.ds(TAIL0, TAIL)])

    return pl.kernel(body, out_type=out_type, mesh=mesh,
                     scratch_types=scratch)


def _sc_hist():
    mesh = plsc.VectorSubcoreMesh(core_axis_name="c", subcore_axis_name="s")
    out_type = jax.ShapeDtypeStruct((NW, N), jnp.float32)
    scratch = [
        pltpu.VMEM((PER_W,), jnp.int32),
        pltpu.VMEM((N,), jnp.float32),
    ]
    cp = pltpu.CompilerParams()
    if "needs_layout_passes" in pltpu.CompilerParams.__dataclass_fields__:
        cp = dataclasses.replace(cp, needs_layout_passes=False)

    def body(dst_hbm, cnt_out, dst_v, cnt_v):
        cid = lax.axis_index("c")
        sid = lax.axis_index("s")
        wid = cid * NS + sid

        pltpu.sync_copy(dst_hbm.at[wid], dst_v)

        @pl.loop(0, N, step=16)
        def _(j):
            cnt_v[pl.ds(j, 16)] = jnp.zeros((16,), jnp.float32)

        ones16 = jnp.ones((16,), jnp.float32)

        @pl.loop(0, PER_W, step=16)
        def _(j):
            plsc.addupdate_scatter(cnt_v, [dst_v[pl.ds(j, 16)]], ones16)

        pltpu.sync_copy(cnt_v, cnt_out.at[wid])

    return pl.kernel(body, out_type=out_type, mesh=mesh,
                     scratch_types=scratch, compiler_params=cp)


def _tc_layer(relu):
    B = 1000

    def body(acc_ref, cnt_ref, x_ref, wl_ref, b_ref, wr_ref, o_ref):
        s = acc_ref[0] + acc_ref[1]
        cnt = jnp.sum(cnt_ref[...], axis=1)
        mean = s * (1.0 / jnp.maximum(cnt, 1.0))[:, None]
        dn = (((1,), (1,)), ((), ()))
        r = (lax.dot_general(mean, wl_ref[...], dn,
                             preferred_element_type=jnp.float32)
             + lax.dot_general(x_ref[...], wr_ref[...], dn,
                               preferred_element_type=jnp.float32)
             + b_ref[...])
        o_ref[...] = jnp.maximum(r, 0.0) if relu else r

    return pl.pallas_call(
        body,
        grid=(N // B,),
        in_specs=[
            pl.BlockSpec((NC, B, D), lambda i: (0, i, 0)),
            pl.BlockSpec((B, NW), lambda i: (i, 0)),
            pl.BlockSpec((B, D), lambda i: (i, 0)),
            pl.BlockSpec((D, D), lambda i: (0, 0)),
            pl.BlockSpec((1, D), lambda i: (0, 0)),
            pl.BlockSpec((D, D), lambda i: (0, 0)),
        ],
        out_specs=pl.BlockSpec((B, D), lambda i: (i, 0)),
        out_shape=jax.ShapeDtypeStruct((N, D), jnp.float32),
    )


@jax.jit
def kernel(x, edge_index, W1_l, b1_l, W1_r, W2_l, b2_l, W2_r):
    ei = edge_index.astype(jnp.int32)
    src = ei[0].reshape(NW, PER_W)
    dst3 = ei[1].reshape(NW, NCH, CH)
    dst2 = ei[1].reshape(NW, PER_W)
    zacc = jnp.zeros((N, D), jnp.float32)
    b1 = b1_l.reshape(1, D)
    b2 = b2_l.reshape(1, D)

    cntp = _sc_hist()(dst2).T
    acc1 = _sc_segsum()(x, src, dst3, zacc)
    h = _tc_layer(True)(acc1, cntp, x, W1_l, b1, W1_r)
    acc2 = _sc_segsum()(h, src, dst3, zacc)
    out = _tc_layer(False)(acc2, cntp, h, W2_l, b2, W2_r)
    return out
